# baseline (device time: 97487 ns/iter reference)
import jax
import jax.numpy as jnp
from jax import lax
from jax.experimental import pallas as pl
from jax.experimental.pallas import tpu as pltpu

N_DEV = 16
HOPS = N_DEV // 2
SEG = 4


def kernel(x, w_mat, scale_x, scale_w):
    m_per, k = x.shape
    _, n_per = w_mat.shape
    m_full = N_DEV * m_per
    seg_rows = m_per // SEG

    scale = (scale_x[0] * scale_w[0]).reshape(1, 1).astype(jnp.float32)

    def body(x_ref, w_ref, scale_ref, out_ref,
             xfull_ref, r_send_sems, r_recv_sems, l_send_sems, l_recv_sems):
        my = lax.axis_index("i")
        left = lax.rem(my + N_DEV - 1, N_DEV)
        right = lax.rem(my + 1, N_DEV)

        barrier_sem = pltpu.get_barrier_semaphore()
        for nbr in (left, right):
            pl.semaphore_signal(
                barrier_sem, inc=1,
                device_id=(nbr,), device_id_type=pl.DeviceIdType.MESH,
            )
        pl.semaphore_wait(barrier_sem, 2)

        s = scale_ref[0, 0]

        def chunk_row(origin):
            return lax.rem(origin + 2 * N_DEV, N_DEV) * m_per

        def seg_slot(origin, seg):
            return pl.ds(chunk_row(origin) + seg * seg_rows, seg_rows)

        def matmul_store(origin, chunk=None):
            if chunk is None:
                chunk = xfull_ref[pl.ds(chunk_row(origin), m_per), :]
            acc = lax.dot_general(
                chunk, w_ref[...],
                dimension_numbers=(((1,), (0,)), ((), ())),
                preferred_element_type=jnp.int32,
            )
            out_ref[pl.ds(chunk_row(origin), m_per), :] = (
                acc.astype(jnp.float32) * s)

        def make_rdma(src, origin, seg, dest, send_sem, recv_sem):
            return pltpu.make_async_remote_copy(
                src_ref=src,
                dst_ref=xfull_ref.at[seg_slot(origin, seg)],
                send_sem=send_sem,
                recv_sem=recv_sem,
                device_id=(dest,),
                device_id_type=pl.DeviceIdType.MESH,
            )

        def r_sends(h):
            return range(SEG) if h < HOPS - 1 else range(SEG // 2)

        def l_sends(h):
            order = range(SEG - 1, -1, -1)
            return order if h < HOPS - 1 else range(SEG - 1, SEG // 2 - 1, -1)

        r_rdma = [[None] * SEG for _ in range(HOPS)]
        l_rdma = [[None] * SEG for _ in range(HOPS)]

        for sg in r_sends(0):
            r_rdma[0][sg] = make_rdma(
                x_ref.at[pl.ds(sg * seg_rows, seg_rows)], my, sg, right,
                r_send_sems.at[0, sg], r_recv_sems.at[0, sg])
            r_rdma[0][sg].start()
        for sg in l_sends(0):
            l_rdma[0][sg] = make_rdma(
                x_ref.at[pl.ds(sg * seg_rows, seg_rows)], my, sg, left,
                l_send_sems.at[0, sg], l_recv_sems.at[0, sg])
            l_rdma[0][sg].start()
        matmul_store(my, chunk=x_ref[...])

        for h in range(1, HOPS):
            r_fwd = set(r_sends(h))
            l_fwd = set(l_sends(h))
            for j in range(SEG):
                rs, ls = j, SEG - 1 - j
                r_rdma[h - 1][rs].wait_recv()
                if rs in r_fwd:
                    r_rdma[h][rs] = make_rdma(
                        xfull_ref.at[seg_slot(my - h, rs)], my - h, rs, right,
                        r_send_sems.at[h, rs], r_recv_sems.at[h, rs])
                    r_rdma[h][rs].start()
                l_rdma[h - 1][ls].wait_recv()
                if ls in l_fwd:
                    l_rdma[h][ls] = make_rdma(
                        xfull_ref.at[seg_slot(my + h, ls)], my + h, ls, left,
                        l_send_sems.at[h, ls], l_recv_sems.at[h, ls])
                    l_rdma[h][ls].start()
            matmul_store(my - h)
            matmul_store(my + h)

        for sg in r_sends(HOPS - 1):
            r_rdma[HOPS - 1][sg].wait_recv()
        for sg in l_sends(HOPS - 1):
            l_rdma[HOPS - 1][sg].wait_recv()
        matmul_store(my - HOPS)

        for h in range(HOPS):
            for sg in range(SEG):
                if r_rdma[h][sg] is not None:
                    r_rdma[h][sg].wait_send()
                if l_rdma[h][sg] is not None:
                    l_rdma[h][sg].wait_send()

    return pl.pallas_call(
        body,
        out_shape=jax.ShapeDtypeStruct((m_full, n_per), jnp.float32),
        in_specs=[
            pl.BlockSpec(memory_space=pltpu.VMEM),
            pl.BlockSpec(memory_space=pltpu.VMEM),
            pl.BlockSpec(memory_space=pltpu.SMEM),
        ],
        out_specs=pl.BlockSpec(memory_space=pltpu.VMEM),
        scratch_shapes=[
            pltpu.VMEM((m_full, k), jnp.int8),
            pltpu.SemaphoreType.DMA((HOPS, SEG)),
            pltpu.SemaphoreType.DMA((HOPS, SEG)),
            pltpu.SemaphoreType.DMA((HOPS, SEG)),
            pltpu.SemaphoreType.DMA((HOPS, SEG)),
        ],
        compiler_params=pltpu.CompilerParams(collective_id=0),
    )(x, w_mat, scale)


# device time: 96880 ns/iter; 1.0063x vs baseline; 1.0063x over previous
import jax
import jax.numpy as jnp
from jax import lax
from jax.experimental import pallas as pl
from jax.experimental.pallas import tpu as pltpu

N_DEV = 16
HOPS = N_DEV // 2
SEG = 2


def kernel(x, w_mat, scale_x, scale_w):
    m_per, k = x.shape
    _, n_per = w_mat.shape
    m_full = N_DEV * m_per
    seg_rows = m_per // SEG

    scale = (scale_x[0] * scale_w[0]).reshape(1, 1).astype(jnp.float32)

    def body(x_ref, w_ref, scale_ref, out_ref,
             xfull_ref, r_send_sems, r_recv_sems, l_send_sems, l_recv_sems):
        my = lax.axis_index("i")
        left = lax.rem(my + N_DEV - 1, N_DEV)
        right = lax.rem(my + 1, N_DEV)

        barrier_sem = pltpu.get_barrier_semaphore()
        for nbr in (left, right):
            pl.semaphore_signal(
                barrier_sem, inc=1,
                device_id=(nbr,), device_id_type=pl.DeviceIdType.MESH,
            )
        pl.semaphore_wait(barrier_sem, 2)

        s = scale_ref[0, 0]

        def chunk_row(origin):
            return lax.rem(origin + 2 * N_DEV, N_DEV) * m_per

        def seg_slot(origin, seg):
            return pl.ds(chunk_row(origin) + seg * seg_rows, seg_rows)

        def matmul_store(origin, chunk=None):
            if chunk is None:
                chunk = xfull_ref[pl.ds(chunk_row(origin), m_per), :]
            acc = lax.dot_general(
                chunk, w_ref[...],
                dimension_numbers=(((1,), (0,)), ((), ())),
                preferred_element_type=jnp.int32,
            )
            out_ref[pl.ds(chunk_row(origin), m_per), :] = (
                acc.astype(jnp.float32) * s)

        def make_rdma(src, origin, seg, dest, send_sem, recv_sem):
            return pltpu.make_async_remote_copy(
                src_ref=src,
                dst_ref=xfull_ref.at[seg_slot(origin, seg)],
                send_sem=send_sem,
                recv_sem=recv_sem,
                device_id=(dest,),
                device_id_type=pl.DeviceIdType.MESH,
            )

        def r_sends(h):
            return range(SEG) if h < HOPS - 1 else range(SEG // 2)

        def l_sends(h):
            order = range(SEG - 1, -1, -1)
            return order if h < HOPS - 1 else range(SEG - 1, SEG // 2 - 1, -1)

        r_rdma = [[None] * SEG for _ in range(HOPS)]
        l_rdma = [[None] * SEG for _ in range(HOPS)]

        for sg in r_sends(0):
            r_rdma[0][sg] = make_rdma(
                x_ref.at[pl.ds(sg * seg_rows, seg_rows)], my, sg, right,
                r_send_sems.at[0, sg], r_recv_sems.at[0, sg])
            r_rdma[0][sg].start()
        for sg in l_sends(0):
            l_rdma[0][sg] = make_rdma(
                x_ref.at[pl.ds(sg * seg_rows, seg_rows)], my, sg, left,
                l_send_sems.at[0, sg], l_recv_sems.at[0, sg])
            l_rdma[0][sg].start()
        matmul_store(my, chunk=x_ref[...])

        for h in range(1, HOPS):
            r_fwd = set(r_sends(h))
            l_fwd = set(l_sends(h))
            for j in range(SEG):
                rs, ls = j, SEG - 1 - j
                r_rdma[h - 1][rs].wait_recv()
                if rs in r_fwd:
                    r_rdma[h][rs] = make_rdma(
                        xfull_ref.at[seg_slot(my - h, rs)], my - h, rs, right,
                        r_send_sems.at[h, rs], r_recv_sems.at[h, rs])
                    r_rdma[h][rs].start()
                l_rdma[h - 1][ls].wait_recv()
                if ls in l_fwd:
                    l_rdma[h][ls] = make_rdma(
                        xfull_ref.at[seg_slot(my + h, ls)], my + h, ls, left,
                        l_send_sems.at[h, ls], l_recv_sems.at[h, ls])
                    l_rdma[h][ls].start()
            matmul_store(my - h)
            matmul_store(my + h)

        for sg in r_sends(HOPS - 1):
            r_rdma[HOPS - 1][sg].wait_recv()
        for sg in l_sends(HOPS - 1):
            l_rdma[HOPS - 1][sg].wait_recv()
        matmul_store(my - HOPS)

        for h in range(HOPS):
            for sg in range(SEG):
                if r_rdma[h][sg] is not None:
                    r_rdma[h][sg].wait_send()
                if l_rdma[h][sg] is not None:
                    l_rdma[h][sg].wait_send()

    return pl.pallas_call(
        body,
        out_shape=jax.ShapeDtypeStruct((m_full, n_per), jnp.float32),
        in_specs=[
            pl.BlockSpec(memory_space=pltpu.VMEM),
            pl.BlockSpec(memory_space=pltpu.VMEM),
            pl.BlockSpec(memory_space=pltpu.SMEM),
        ],
        out_specs=pl.BlockSpec(memory_space=pltpu.VMEM),
        scratch_shapes=[
            pltpu.VMEM((m_full, k), jnp.int8),
            pltpu.SemaphoreType.DMA((HOPS, SEG)),
            pltpu.SemaphoreType.DMA((HOPS, SEG)),
            pltpu.SemaphoreType.DMA((HOPS, SEG)),
            pltpu.SemaphoreType.DMA((HOPS, SEG)),
        ],
        compiler_params=pltpu.CompilerParams(collective_id=0),
    )(x, w_mat, scale)


# device time: 96836 ns/iter; 1.0067x vs baseline; 1.0005x over previous
import jax
import jax.numpy as jnp
from jax import lax
from jax.experimental import pallas as pl
from jax.experimental.pallas import tpu as pltpu

N_DEV = 16
HOPS = N_DEV // 2
SEG = 2


def kernel(x, w_mat, scale_x, scale_w):
    m_per, k = x.shape
    _, n_per = w_mat.shape
    m_full = N_DEV * m_per
    seg_rows = m_per // SEG

    scale = (scale_x[0] * scale_w[0]).reshape(1, 1).astype(jnp.float32)

    def body(x_ref, w_ref, scale_ref, out_ref,
             xfull_ref, r_send_sems, r_recv_sems, l_send_sems, l_recv_sems):
        my = lax.axis_index("i")
        left = lax.rem(my + N_DEV - 1, N_DEV)
        right = lax.rem(my + 1, N_DEV)

        barrier_sem = pltpu.get_barrier_semaphore()
        for nbr in (left, right):
            pl.semaphore_signal(
                barrier_sem, inc=1,
                device_id=(nbr,), device_id_type=pl.DeviceIdType.MESH,
            )
        pl.semaphore_wait(barrier_sem, 2)

        s = scale_ref[0, 0]

        def chunk_row(origin):
            return lax.rem(origin + 2 * N_DEV, N_DEV) * m_per

        def seg_slot(origin, seg):
            return pl.ds(chunk_row(origin) + seg * seg_rows, seg_rows)

        def matmul_store(origin, chunk=None):
            if chunk is None:
                chunk = xfull_ref[pl.ds(chunk_row(origin), m_per), :]
            acc = lax.dot_general(
                chunk, w_ref[...],
                dimension_numbers=(((1,), (0,)), ((), ())),
                preferred_element_type=jnp.int32,
            )
            out_ref[pl.ds(chunk_row(origin), m_per), :] = (
                acc.astype(jnp.float32) * s)

        def make_rdma(src, origin, seg, dest, send_sem, recv_sem):
            return pltpu.make_async_remote_copy(
                src_ref=src,
                dst_ref=xfull_ref.at[seg_slot(origin, seg)],
                send_sem=send_sem,
                recv_sem=recv_sem,
                device_id=(dest,),
                device_id_type=pl.DeviceIdType.MESH,
            )

        def r_sends(h):
            return range(SEG) if h < HOPS - 1 else range(SEG // 2)

        def l_sends(h):
            order = range(SEG - 1, -1, -1)
            return order if h < HOPS - 1 else range(SEG - 1, SEG // 2 - 1, -1)

        r_rdma = [[None] * SEG for _ in range(HOPS)]
        l_rdma = [[None] * SEG for _ in range(HOPS)]

        for sg in r_sends(0):
            r_rdma[0][sg] = make_rdma(
                x_ref.at[pl.ds(sg * seg_rows, seg_rows)], my, sg, right,
                r_send_sems.at[0, sg], r_recv_sems.at[0, sg])
            r_rdma[0][sg].start()
        for sg in l_sends(0):
            l_rdma[0][sg] = make_rdma(
                x_ref.at[pl.ds(sg * seg_rows, seg_rows)], my, sg, left,
                l_send_sems.at[0, sg], l_recv_sems.at[0, sg])
            l_rdma[0][sg].start()
        matmul_store(my, chunk=x_ref[...])

        at_wrap = my == 0

        for h in range(1, HOPS):
            for rs in r_sends(h):
                r_rdma[h][rs] = make_rdma(
                    xfull_ref.at[seg_slot(my - h, rs)], my - h, rs, right,
                    r_send_sems.at[h, rs], r_recv_sems.at[h, rs])
            for ls in l_sends(h):
                l_rdma[h][ls] = make_rdma(
                    xfull_ref.at[seg_slot(my + h, ls)], my + h, ls, left,
                    l_send_sems.at[h, ls], l_recv_sems.at[h, ls])

            def run_hop(h, right_first):
                for j in range(SEG):
                    rs, ls = j, SEG - 1 - j
                    steps = [
                        (r_rdma[h - 1][rs], r_rdma[h][rs]),
                        (l_rdma[h - 1][ls], l_rdma[h][ls]),
                    ]
                    if not right_first:
                        steps.reverse()
                    for prev, nxt in steps:
                        prev.wait_recv()
                        if nxt is not None:
                            nxt.start()

            @pl.when(at_wrap)
            def _(h=h):
                run_hop(h, right_first=False)

            @pl.when(jnp.logical_not(at_wrap))
            def _(h=h):
                run_hop(h, right_first=True)

            matmul_store(my - h)
            matmul_store(my + h)

        for sg in r_sends(HOPS - 1):
            r_rdma[HOPS - 1][sg].wait_recv()
        for sg in l_sends(HOPS - 1):
            l_rdma[HOPS - 1][sg].wait_recv()
        matmul_store(my - HOPS)

        for h in range(HOPS):
            for sg in range(SEG):
                if r_rdma[h][sg] is not None:
                    r_rdma[h][sg].wait_send()
                if l_rdma[h][sg] is not None:
                    l_rdma[h][sg].wait_send()

    return pl.pallas_call(
        body,
        out_shape=jax.ShapeDtypeStruct((m_full, n_per), jnp.float32),
        in_specs=[
            pl.BlockSpec(memory_space=pltpu.VMEM),
            pl.BlockSpec(memory_space=pltpu.VMEM),
            pl.BlockSpec(memory_space=pltpu.SMEM),
        ],
        out_specs=pl.BlockSpec(memory_space=pltpu.VMEM),
        scratch_shapes=[
            pltpu.VMEM((m_full, k), jnp.int8),
            pltpu.SemaphoreType.DMA((HOPS, SEG)),
            pltpu.SemaphoreType.DMA((HOPS, SEG)),
            pltpu.SemaphoreType.DMA((HOPS, SEG)),
            pltpu.SemaphoreType.DMA((HOPS, SEG)),
        ],
        compiler_params=pltpu.CompilerParams(collective_id=0),
    )(x, w_mat, scale)
